# Initial kernel scaffold; baseline (speedup 1.0000x reference)
import jax
import jax.numpy as jnp
from jax.experimental import pallas as pl


def kernel(inputs_d, inputs_q, mask_d, mask_q, emb_weight):
    raise NotImplementedError("write your pallas kernel here")


# speedup vs baseline: 3.8323x; 3.8323x over previous
import functools

import jax
import jax.numpy as jnp
from jax import lax
from jax.experimental import pallas as pl
from jax.experimental.pallas import tpu as pltpu
from jax.experimental.pallas import tpu_sc as plsc

_EPS_NORM = 1e-12
_EPS_COS = 1e-8
_LANES = 16


def _normalize_body(w_ref, out_ref):
    x = w_ref[...]
    s = jnp.sum(x * x, axis=1, keepdims=True)
    out_ref[...] = x / jnp.maximum(jnp.sqrt(s), _EPS_NORM)


def _normalize_table(w, rows_per_block=1000):
    v, d = w.shape
    assert v % rows_per_block == 0
    return pl.pallas_call(
        _normalize_body,
        grid=(v // rows_per_block,),
        in_specs=[pl.BlockSpec((rows_per_block, d), lambda i: (i, 0))],
        out_specs=pl.BlockSpec((rows_per_block, d), lambda i: (i, 0)),
        out_shape=jax.ShapeDtypeStruct((v, d), jnp.float32),
    )(w)


def _cos_body(q_ref, d_ref, out_ref):
    q = q_ref[...]
    d = d_ref[...]
    dot = jnp.sum(q * d, axis=1)
    nq = jnp.maximum(jnp.sqrt(jnp.sum(q * q, axis=1)), _EPS_COS)
    nd = jnp.maximum(jnp.sqrt(jnp.sum(d * d, axis=1)), _EPS_COS)
    out_ref[...] = dot / (nq * nd)


def _cosine(maxq, maxd, rows_per_block=512):
    b, d = maxq.shape
    assert b % rows_per_block == 0
    return pl.pallas_call(
        _cos_body,
        grid=(b // rows_per_block,),
        in_specs=[pl.BlockSpec((rows_per_block, d), lambda i: (i, 0))] * 2,
        out_specs=pl.BlockSpec((rows_per_block,), lambda i: (i,)),
        out_shape=jax.ShapeDtypeStruct((b,), jnp.float32),
    )(maxq, maxd)


def _make_sc_maxpool(bsz, d, ldp, lqp, n_cores, n_subcores, chunk,
                     interpret=False):
    n_workers = n_cores * n_subcores
    per_w = bsz // n_workers
    assert per_w % chunk == 0
    n_chunks = per_w // chunk
    half = ldp // 2
    nj = d // _LANES
    mesh = plsc.VectorSubcoreMesh(
        core_axis_name="c", subcore_axis_name="s",
        num_cores=n_cores, num_subcores=n_subcores)

    @functools.partial(
        pl.kernel,
        out_type=(
            jax.ShapeDtypeStruct((bsz, d), jnp.float32),
            jax.ShapeDtypeStruct((bsz, d), jnp.float32),
        ),
        mesh=mesh,
        interpret=interpret,
        scratch_types=[
            pltpu.VMEM((chunk, 2, half), jnp.int32),
            pltpu.VMEM((chunk, lqp), jnp.int32),
            pltpu.VMEM((chunk, ldp), jnp.float32),
            pltpu.VMEM((chunk, lqp), jnp.float32),
            pltpu.VMEM((ldp, d), jnp.float32),
            pltpu.VMEM((lqp, d), jnp.float32),
            pltpu.VMEM((chunk, d), jnp.float32),
            pltpu.VMEM((chunk, d), jnp.float32),
            pltpu.SemaphoreType.DMA,
        ],
    )
    def sc_kernel(table, idxd, idxq, maskd, maskq, outq, outd,
                  idxd_v, idxq_v, maskd_v, maskq_v, rowsd_v, rowsq_v,
                  oq_v, od_v, sem):
        wid = lax.axis_index("s") * n_cores + lax.axis_index("c")
        w_base = wid * per_w

        def chunk_body(ci, carry):
            base = w_base + ci * chunk
            pltpu.sync_copy(idxd.at[pl.ds(base, chunk)], idxd_v)
            pltpu.sync_copy(idxq.at[pl.ds(base, chunk)], idxq_v)
            pltpu.sync_copy(maskd.at[pl.ds(base, chunk)], maskd_v)
            pltpu.sync_copy(maskq.at[pl.ds(base, chunk)], maskq_v)

            def batch_body(bl, carry):
                cp0 = pltpu.async_copy(
                    table.at[idxd_v.at[bl, 0]], rowsd_v.at[pl.ds(0, half)], sem)
                cp1 = pltpu.async_copy(
                    table.at[idxd_v.at[bl, 1]], rowsd_v.at[pl.ds(half, half)], sem)
                cp2 = pltpu.async_copy(table.at[idxq_v.at[bl]], rowsq_v, sem)
                cp0.wait()
                cp1.wait()
                cp2.wait()

                neg = jnp.full((_LANES,), -jnp.inf, jnp.float32)

                def row_max(rows_ref, mask_ref, nrows):
                    def gbody(g, acc):
                        mvec = mask_ref[bl, pl.ds(g * _LANES, _LANES)]
                        for i in range(_LANES):
                            l = g * _LANES + i
                            m = mvec[i]
                            acc = tuple(
                                jnp.maximum(
                                    acc[j],
                                    rows_ref[l, pl.ds(j * _LANES, _LANES)] * m)
                                for j in range(nj))
                        return acc
                    return lax.fori_loop(0, nrows // _LANES, gbody, (neg,) * nj)

                accd = row_max(rowsd_v, maskd_v, ldp)
                accq = row_max(rowsq_v, maskq_v, lqp)
                for j in range(nj):
                    od_v[bl, pl.ds(j * _LANES, _LANES)] = accd[j]
                    oq_v[bl, pl.ds(j * _LANES, _LANES)] = accq[j]
                return carry

            lax.fori_loop(0, chunk, batch_body, 0)
            pltpu.sync_copy(oq_v, outq.at[pl.ds(base, chunk)])
            pltpu.sync_copy(od_v, outd.at[pl.ds(base, chunk)])
            return carry

        lax.fori_loop(0, n_chunks, chunk_body, 0)

    return sc_kernel


def kernel(inputs_d, inputs_q, mask_d, mask_q, emb_weight):
    bsz, ld = inputs_d.shape
    _, lq = inputs_q.shape
    v, d = emb_weight.shape

    ldp = ld + (-ld) % 16
    pad_d = ldp - ld
    pad_q = (-lq) % 16
    lqp = lq + pad_q
    idx_d = jnp.concatenate(
        [inputs_d, inputs_d[:, :pad_d]], axis=1).astype(jnp.int32)
    mk_d = jnp.concatenate([mask_d, mask_d[:, :pad_d]], axis=1)
    if pad_q:
        idx_q = jnp.concatenate(
            [inputs_q, inputs_q[:, :pad_q]], axis=1).astype(jnp.int32)
        mk_q = jnp.concatenate([mask_q, mask_q[:, :pad_q]], axis=1)
    else:
        idx_q, mk_q = inputs_q.astype(jnp.int32), mask_q
    idx_d = idx_d.reshape(bsz, 2, ldp // 2)

    table_n = _normalize_table(emb_weight)

    info = plsc.get_sparse_core_info()
    sc = _make_sc_maxpool(bsz, d, ldp, lqp, info.num_cores,
                          info.num_subcores, chunk=32)
    maxq, maxd = sc(table_n, idx_d, idx_q, mk_d, mk_q)
    return _cosine(maxq, maxd)


# speedup vs baseline: 5.1012x; 1.3311x over previous
import functools

import jax
import jax.numpy as jnp
from jax import lax
from jax.experimental import pallas as pl
from jax.experimental.pallas import tpu as pltpu
from jax.experimental.pallas import tpu_sc as plsc

_EPS_NORM = 1e-12
_EPS_COS = 1e-8
_LANES = 16


def _normalize_body(w_ref, out_ref):
    x = w_ref[...]
    s = jnp.sum(x * x, axis=1, keepdims=True)
    out_ref[...] = x / jnp.maximum(jnp.sqrt(s), _EPS_NORM)


def _normalize_table(w, rows_per_block=1000):
    v, d = w.shape
    assert v % rows_per_block == 0
    return pl.pallas_call(
        _normalize_body,
        grid=(v // rows_per_block,),
        in_specs=[pl.BlockSpec((rows_per_block, d), lambda i: (i, 0))],
        out_specs=pl.BlockSpec((rows_per_block, d), lambda i: (i, 0)),
        out_shape=jax.ShapeDtypeStruct((v, d), jnp.float32),
    )(w)


def _cos_body(q_ref, d_ref, out_ref):
    q = q_ref[...]
    d = d_ref[...]
    dot = jnp.sum(q * d, axis=1)
    nq = jnp.maximum(jnp.sqrt(jnp.sum(q * q, axis=1)), _EPS_COS)
    nd = jnp.maximum(jnp.sqrt(jnp.sum(d * d, axis=1)), _EPS_COS)
    out_ref[...] = dot / (nq * nd)


def _cosine(maxq, maxd, rows_per_block=512):
    b, d = maxq.shape
    assert b % rows_per_block == 0
    return pl.pallas_call(
        _cos_body,
        grid=(b // rows_per_block,),
        in_specs=[pl.BlockSpec((rows_per_block, d), lambda i: (i, 0))] * 2,
        out_specs=pl.BlockSpec((rows_per_block,), lambda i: (i,)),
        out_shape=jax.ShapeDtypeStruct((b,), jnp.float32),
    )(maxq, maxd)


def _make_sc_maxpool(bsz, d, ldp, lqp, n_cores, n_subcores, chunk,
                     interpret=False):
    n_workers = n_cores * n_subcores
    per_w = bsz // n_workers
    assert per_w % chunk == 0
    n_chunks = per_w // chunk
    half = ldp // 2
    nj = d // _LANES
    mesh = plsc.VectorSubcoreMesh(
        core_axis_name="c", subcore_axis_name="s",
        num_cores=n_cores, num_subcores=n_subcores)

    @functools.partial(
        pl.kernel,
        out_type=(
            jax.ShapeDtypeStruct((bsz, d), jnp.float32),
            jax.ShapeDtypeStruct((bsz, d), jnp.float32),
        ),
        mesh=mesh,
        interpret=interpret,
        scratch_types=[
            pltpu.VMEM((chunk, 2, half), jnp.int32),
            pltpu.VMEM((chunk, lqp), jnp.int32),
            pltpu.VMEM((chunk, ldp), jnp.float32),
            pltpu.VMEM((chunk, lqp), jnp.float32),
            pltpu.VMEM((ldp, d), jnp.float32),
            pltpu.VMEM((ldp, d), jnp.float32),
            pltpu.VMEM((lqp, d), jnp.float32),
            pltpu.VMEM((lqp, d), jnp.float32),
            pltpu.VMEM((chunk, d), jnp.float32),
            pltpu.VMEM((chunk, d), jnp.float32),
            pltpu.SemaphoreType.DMA,
            pltpu.SemaphoreType.DMA,
        ],
    )
    def sc_kernel(table, idxd, idxq, maskd, maskq, outq, outd,
                  idxd_v, idxq_v, maskd_v, maskq_v,
                  rd0, rd1, rq0, rq1, oq_v, od_v, sem0, sem1):
        wid = lax.axis_index("s") * n_cores + lax.axis_index("c")
        w_base = wid * per_w

        def copies(bl, rd, rq, sem):
            return (
                pltpu.make_async_copy(
                    table.at[idxd_v.at[bl, 0]], rd.at[pl.ds(0, half)], sem),
                pltpu.make_async_copy(
                    table.at[idxd_v.at[bl, 1]], rd.at[pl.ds(half, half)], sem),
                pltpu.make_async_copy(table.at[idxq_v.at[bl]], rq, sem),
            )

        def fire(bl, rd, rq, sem):
            for cp in copies(bl, rd, rq, sem):
                cp.start()

        def wait(bl, rd, rq, sem):
            for cp in copies(bl, rd, rq, sem):
                cp.wait()

        neg = jnp.full((_LANES,), -jnp.inf, jnp.float32)

        def compute(bl, rd, rq):
            def row_max(rows_ref, mask_ref, nrows):
                def gbody(g, acc):
                    mvec = mask_ref[bl, pl.ds(g * _LANES, _LANES)]
                    for i in range(_LANES):
                        l = g * _LANES + i
                        m = mvec[i]
                        acc = tuple(
                            jnp.maximum(
                                acc[j],
                                rows_ref[l, pl.ds(j * _LANES, _LANES)] * m)
                            for j in range(nj))
                    return acc
                return lax.fori_loop(0, nrows // _LANES, gbody, (neg,) * nj)

            accd = row_max(rd, maskd_v, ldp)
            accq = row_max(rq, maskq_v, lqp)
            for j in range(nj):
                od_v[bl, pl.ds(j * _LANES, _LANES)] = accd[j]
                oq_v[bl, pl.ds(j * _LANES, _LANES)] = accq[j]

        def chunk_body(ci, carry):
            base = w_base + ci * chunk
            pltpu.sync_copy(idxd.at[pl.ds(base, chunk)], idxd_v)
            pltpu.sync_copy(idxq.at[pl.ds(base, chunk)], idxq_v)
            pltpu.sync_copy(maskd.at[pl.ds(base, chunk)], maskd_v)
            pltpu.sync_copy(maskq.at[pl.ds(base, chunk)], maskq_v)

            fire(0, rd0, rq0, sem0)

            def step(k, carry):
                b0 = 2 * k
                b1 = b0 + 1
                fire(b1, rd1, rq1, sem1)
                wait(b0, rd0, rq0, sem0)
                compute(b0, rd0, rq0)

                @pl.when(b1 + 1 < chunk)
                def _():
                    fire(b1 + 1, rd0, rq0, sem0)
                wait(b1, rd1, rq1, sem1)
                compute(b1, rd1, rq1)
                return carry

            lax.fori_loop(0, chunk // 2, step, 0)
            pltpu.sync_copy(oq_v, outq.at[pl.ds(base, chunk)])
            pltpu.sync_copy(od_v, outd.at[pl.ds(base, chunk)])
            return carry

        lax.fori_loop(0, n_chunks, chunk_body, 0)

    return sc_kernel


def kernel(inputs_d, inputs_q, mask_d, mask_q, emb_weight):
    bsz, ld = inputs_d.shape
    _, lq = inputs_q.shape
    v, d = emb_weight.shape

    ldp = ld + (-ld) % 16
    pad_d = ldp - ld
    pad_q = (-lq) % 16
    lqp = lq + pad_q
    idx_d = jnp.concatenate(
        [inputs_d, inputs_d[:, :pad_d]], axis=1).astype(jnp.int32)
    mk_d = jnp.concatenate([mask_d, mask_d[:, :pad_d]], axis=1)
    if pad_q:
        idx_q = jnp.concatenate(
            [inputs_q, inputs_q[:, :pad_q]], axis=1).astype(jnp.int32)
        mk_q = jnp.concatenate([mask_q, mask_q[:, :pad_q]], axis=1)
    else:
        idx_q, mk_q = inputs_q.astype(jnp.int32), mask_q
    idx_d = idx_d.reshape(bsz, 2, ldp // 2)

    table_n = _normalize_table(emb_weight)

    info = plsc.get_sparse_core_info()
    sc = _make_sc_maxpool(bsz, d, ldp, lqp, info.num_cores,
                          info.num_subcores, chunk=32)
    maxq, maxd = sc(table_n, idx_d, idx_q, mk_d, mk_q)
    return _cosine(maxq, maxd)


# speedup vs baseline: 5.2948x; 1.0380x over previous
import functools

import jax
import jax.numpy as jnp
from jax import lax
from jax.experimental import pallas as pl
from jax.experimental.pallas import tpu as pltpu
from jax.experimental.pallas import tpu_sc as plsc

_EPS_NORM = 1e-12
_EPS_COS = 1e-8
_LANES = 16


def _normalize_body(w_ref, out_ref):
    x = w_ref[...]
    s = jnp.sum(x * x, axis=1, keepdims=True)
    out_ref[...] = x / jnp.maximum(jnp.sqrt(s), _EPS_NORM)


def _normalize_table(w, rows_per_block=1000):
    v, d = w.shape
    assert v % rows_per_block == 0
    return pl.pallas_call(
        _normalize_body,
        grid=(v // rows_per_block,),
        in_specs=[pl.BlockSpec((rows_per_block, d), lambda i: (i, 0))],
        out_specs=pl.BlockSpec((rows_per_block, d), lambda i: (i, 0)),
        out_shape=jax.ShapeDtypeStruct((v, d), jnp.float32),
    )(w)


def _cos_body(q_ref, d_ref, out_ref):
    q = q_ref[...]
    d = d_ref[...]
    dot = jnp.sum(q * d, axis=1)
    nq = jnp.maximum(jnp.sqrt(jnp.sum(q * q, axis=1)), _EPS_COS)
    nd = jnp.maximum(jnp.sqrt(jnp.sum(d * d, axis=1)), _EPS_COS)
    out_ref[...] = dot / (nq * nd)


def _cosine(maxq, maxd, rows_per_block=512):
    b, d = maxq.shape
    assert b % rows_per_block == 0
    return pl.pallas_call(
        _cos_body,
        grid=(b // rows_per_block,),
        in_specs=[pl.BlockSpec((rows_per_block, d), lambda i: (i, 0))] * 2,
        out_specs=pl.BlockSpec((rows_per_block,), lambda i: (i,)),
        out_shape=jax.ShapeDtypeStruct((b,), jnp.float32),
    )(maxq, maxd)


def _make_sc_maxpool(bsz, d, ldp, lqp, n_cores, n_subcores, chunk,
                     interpret=False):
    n_workers = n_cores * n_subcores
    per_w = bsz // n_workers
    assert per_w % chunk == 0
    n_chunks = per_w // chunk
    lt = ldp + lqp
    half = lt // 2
    assert half % 8 == 0 and half <= 128 and ldp % _LANES == 0
    nj = d // _LANES
    ngd = ldp // _LANES
    ngt = lt // _LANES
    mesh = plsc.VectorSubcoreMesh(
        core_axis_name="c", subcore_axis_name="s",
        num_cores=n_cores, num_subcores=n_subcores)

    @functools.partial(
        pl.kernel,
        out_type=(
            jax.ShapeDtypeStruct((bsz, d), jnp.float32),
            jax.ShapeDtypeStruct((bsz, d), jnp.float32),
        ),
        mesh=mesh,
        interpret=interpret,
        scratch_types=[
            pltpu.VMEM((chunk, 2, half), jnp.int32),
            pltpu.VMEM((chunk, lt), jnp.float32),
            pltpu.VMEM((lt, d), jnp.float32),
            pltpu.VMEM((lt, d), jnp.float32),
            pltpu.VMEM((chunk, d), jnp.float32),
            pltpu.VMEM((chunk, d), jnp.float32),
            pltpu.SemaphoreType.DMA,
            pltpu.SemaphoreType.DMA,
        ],
    )
    def sc_kernel(table, idx, mask, outq, outd,
                  idx_v, mask_v, r0, r1, oq_v, od_v, sem0, sem1):
        wid = lax.axis_index("s") * n_cores + lax.axis_index("c")
        w_base = wid * per_w

        def copies(bl, rows, sem):
            return (
                pltpu.make_async_copy(
                    table.at[idx_v.at[bl, 0]], rows.at[pl.ds(0, half)], sem),
                pltpu.make_async_copy(
                    table.at[idx_v.at[bl, 1]], rows.at[pl.ds(half, half)],
                    sem),
            )

        def fire(bl, rows, sem):
            for cp in copies(bl, rows, sem):
                cp.start()

        def wait(bl, rows, sem):
            for cp in copies(bl, rows, sem):
                cp.wait()

        neg = jnp.full((_LANES,), -jnp.inf, jnp.float32)

        def row_max(bl, rows_ref, g_lo, g_hi):
            def gbody(g, acc):
                mvec = mask_v[bl, pl.ds(g * _LANES, _LANES)]
                for i in range(_LANES):
                    l = g * _LANES + i
                    m = mvec[i]
                    acc = tuple(
                        jnp.maximum(
                            acc[j],
                            rows_ref[l, pl.ds(j * _LANES, _LANES)] * m)
                        for j in range(nj))
                return acc
            return lax.fori_loop(g_lo, g_hi, gbody, (neg,) * nj)

        def compute(bl, rows):
            accd = row_max(bl, rows, 0, ngd)
            accq = row_max(bl, rows, ngd, ngt)
            for j in range(nj):
                od_v[bl, pl.ds(j * _LANES, _LANES)] = accd[j]
                oq_v[bl, pl.ds(j * _LANES, _LANES)] = accq[j]

        def chunk_body(ci, carry):
            base = w_base + ci * chunk
            pltpu.sync_copy(idx.at[pl.ds(base, chunk)], idx_v)
            pltpu.sync_copy(mask.at[pl.ds(base, chunk)], mask_v)

            fire(0, r0, sem0)

            def step(k, carry):
                b0 = 2 * k
                b1 = b0 + 1
                fire(b1, r1, sem1)
                wait(b0, r0, sem0)
                compute(b0, r0)

                @pl.when(b1 + 1 < chunk)
                def _():
                    fire(b1 + 1, r0, sem0)
                wait(b1, r1, sem1)
                compute(b1, r1)
                return carry

            lax.fori_loop(0, chunk // 2, step, 0)
            pltpu.sync_copy(oq_v, outq.at[pl.ds(base, chunk)])
            pltpu.sync_copy(od_v, outd.at[pl.ds(base, chunk)])
            return carry

        lax.fori_loop(0, n_chunks, chunk_body, 0)

    return sc_kernel


def kernel(inputs_d, inputs_q, mask_d, mask_q, emb_weight):
    bsz, ld = inputs_d.shape
    _, lq = inputs_q.shape
    v, d = emb_weight.shape

    ldp = ld + (-ld) % 16
    pad_d = ldp - ld
    pad_q = (-lq) % 16
    lqp = lq + pad_q
    idx = jnp.concatenate(
        [inputs_d, inputs_d[:, :pad_d], inputs_q, inputs_q[:, :pad_q]],
        axis=1).astype(jnp.int32)
    mask = jnp.concatenate(
        [mask_d, mask_d[:, :pad_d], mask_q, mask_q[:, :pad_q]], axis=1)
    idx = idx.reshape(bsz, 2, (ldp + lqp) // 2)

    table_n = _normalize_table(emb_weight)

    info = plsc.get_sparse_core_info()
    sc = _make_sc_maxpool(bsz, d, ldp, lqp, info.num_cores,
                          info.num_subcores, chunk=32)
    maxq, maxd = sc(table_n, idx, mask)
    return _cosine(maxq, maxd)
